# baseline (speedup 1.0000x reference)
import jax
import jax.numpy as jnp
from jax.experimental import pallas as pl
from jax.experimental.pallas import tpu as pltpu

SIZE = 262144
EMBED_DIM = 128
BATCH = 4096
CHUNK = 16384
NCHUNK = SIZE // CHUNK
NBUF = 7


def _body(ptr_ref, q_hbm, emb_hbm, out_hbm, optr_ref,
          bufs, emb_buf, sem_in, sem_out, sem_emb):
    raw_ptr = ptr_ref[0]
    ptr = jnp.clip(raw_ptr, 0, SIZE - BATCH)
    optr_ref[0] = (raw_ptr + BATCH) % SIZE

    k0 = ptr // CHUNK

    RAMP = CHUNK // 4
    SCHEDULE = (
        [(2 * CHUNK + j * RAMP, RAMP) for j in range(4)]
        + [(3 * CHUNK, CHUNK)]
        + [(0, CHUNK), (CHUNK, CHUNK)]
        + [((4 + j) * CHUNK, CHUNK) for j in range(NCHUNK - 5)]
        + [((NCHUNK - 1) * CHUNK + j * RAMP, RAMP) for j in range(4)]
    )
    NPOS = len(SCHEDULE)
    WPOS = 5

    def hbm_start(p):
        rel, _ = SCHEDULE[p]
        return (k0 * CHUNK + rel) % SIZE

    def in_copy(p):
        _, n = SCHEDULE[p]
        s = p % NBUF
        return pltpu.make_async_copy(
            q_hbm.at[pl.ds(hbm_start(p), n)],
            bufs.at[pl.ds(s * CHUNK, n)], sem_in.at[s])

    def out_copy(p):
        _, n = SCHEDULE[p]
        s = p % NBUF
        return pltpu.make_async_copy(
            bufs.at[pl.ds(s * CHUNK, n)],
            out_hbm.at[pl.ds(hbm_start(p), n)], sem_out.at[s])

    emb_in = pltpu.make_async_copy(emb_hbm, emb_buf, sem_emb)
    off = ptr - k0 * CHUNK
    splice = pltpu.make_async_copy(
        emb_buf, bufs.at[pl.ds(WPOS * CHUNK + off, BATCH)], sem_emb)

    emb_in.start()
    for p in range(NBUF):
        in_copy(p).start()

    in_waited = set()
    out_waited = set()

    def ensure_in(p):
        if p not in in_waited:
            in_copy(p).wait()
            in_waited.add(p)

    def ensure_out(p):
        if p not in out_waited:
            out_copy(p).wait()
            out_waited.add(p)

    for p in range(NPOS):
        ensure_in(p)
        if p == WPOS:
            ensure_in(p + 1)
            emb_in.wait()
            splice.start()
            splice.wait()
        out_copy(p).start()
        nxt = p + NBUF
        if nxt < NPOS:
            ensure_out(nxt - NBUF)
            in_copy(nxt).start()
    for p in range(NPOS):
        ensure_out(p)


def kernel(embed_queue, queue_ptr, embeddings):
    new_queue, new_ptr = pl.pallas_call(
        _body,
        in_specs=[
            pl.BlockSpec(memory_space=pltpu.SMEM),
            pl.BlockSpec(memory_space=pl.ANY),
            pl.BlockSpec(memory_space=pl.ANY),
        ],
        out_specs=[
            pl.BlockSpec(memory_space=pl.ANY),
            pl.BlockSpec(memory_space=pltpu.SMEM),
        ],
        out_shape=[
            jax.ShapeDtypeStruct((SIZE, EMBED_DIM), jnp.float32),
            jax.ShapeDtypeStruct((1,), jnp.int32),
        ],
        scratch_shapes=[
            pltpu.VMEM((NBUF * CHUNK, EMBED_DIM), jnp.float32),
            pltpu.VMEM((BATCH, EMBED_DIM), jnp.float32),
            pltpu.SemaphoreType.DMA((NBUF,)),
            pltpu.SemaphoreType.DMA((NBUF,)),
            pltpu.SemaphoreType.DMA,
        ],
    )(queue_ptr, embed_queue, embeddings)
    return new_queue, new_ptr


# speedup vs baseline: 1.0100x; 1.0100x over previous
import jax
import jax.numpy as jnp
from jax.experimental import pallas as pl
from jax.experimental.pallas import tpu as pltpu

SIZE = 262144
EMBED_DIM = 128
BATCH = 4096
CHUNK = 16384
NCHUNK = SIZE // CHUNK
NBUF = 7


def _body(ptr_ref, q_hbm, emb_hbm, out_hbm, optr_ref,
          bufs, emb_buf, sem_in, sem_out, sem_emb, sem_fill):
    raw_ptr = ptr_ref[0]
    ptr = jnp.clip(raw_ptr, 0, SIZE - BATCH)
    optr_ref[0] = (raw_ptr + BATCH) % SIZE

    k0 = ptr // CHUNK

    SCHEDULE = (
        [(2 * CHUNK, CHUNK), (3 * CHUNK, CHUNK)]
        + [(0, CHUNK), (CHUNK, CHUNK)]
        + [((4 + j) * CHUNK, CHUNK) for j in range(NCHUNK - 4)]
    )
    NPOS = len(SCHEDULE)
    WPOS = 2

    def npiece(p):
        return 4 if p in (0, NPOS - 1) else 1

    def hbm_start(p):
        rel, _ = SCHEDULE[p]
        return (k0 * CHUNK + rel) % SIZE

    def in_copy(p, k=0):
        _, n = SCHEDULE[p]
        split = npiece(p) > 1
        n //= npiece(p)
        s = p % NBUF
        sem = sem_fill.at[k] if split else sem_in.at[s]
        return pltpu.make_async_copy(
            q_hbm.at[pl.ds(hbm_start(p) + k * n, n)],
            bufs.at[pl.ds(s * CHUNK + k * n, n)], sem)

    def out_copy(p, k=0):
        _, n = SCHEDULE[p]
        n //= npiece(p)
        s = p % NBUF
        return pltpu.make_async_copy(
            bufs.at[pl.ds(s * CHUNK + k * n, n)],
            out_hbm.at[pl.ds(hbm_start(p) + k * n, n)], sem_out.at[s])

    emb_in = pltpu.make_async_copy(emb_hbm, emb_buf, sem_emb)
    off = ptr - k0 * CHUNK
    splice = pltpu.make_async_copy(
        emb_buf, bufs.at[pl.ds(WPOS * CHUNK + off, BATCH)], sem_emb)

    emb_in.start()
    for p in range(NBUF):
        for k in range(npiece(p)):
            in_copy(p, k).start()

    in_waited = set()
    out_waited = set()

    def ensure_in(p):
        if p not in in_waited:
            for k in range(npiece(p)):
                in_copy(p, k).wait()
            in_waited.add(p)

    def ensure_out(p):
        if p not in out_waited:
            for k in range(npiece(p)):
                out_copy(p, k).wait()
            out_waited.add(p)

    for p in range(NPOS):
        if npiece(p) == 1:
            ensure_in(p)
            if p == WPOS:
                ensure_in(p + 1)
                emb_in.wait()
                splice.start()
                splice.wait()
            out_copy(p).start()
        else:
            for k in range(npiece(p)):
                in_copy(p, k).wait()
                out_copy(p, k).start()
            in_waited.add(p)
        nxt = p + NBUF
        if nxt < NPOS:
            ensure_out(nxt - NBUF)
            for k in range(npiece(nxt)):
                in_copy(nxt, k).start()
    for p in range(NPOS):
        ensure_out(p)


def kernel(embed_queue, queue_ptr, embeddings):
    new_queue, new_ptr = pl.pallas_call(
        _body,
        in_specs=[
            pl.BlockSpec(memory_space=pltpu.SMEM),
            pl.BlockSpec(memory_space=pl.ANY),
            pl.BlockSpec(memory_space=pl.ANY),
        ],
        out_specs=[
            pl.BlockSpec(memory_space=pl.ANY),
            pl.BlockSpec(memory_space=pltpu.SMEM),
        ],
        out_shape=[
            jax.ShapeDtypeStruct((SIZE, EMBED_DIM), jnp.float32),
            jax.ShapeDtypeStruct((1,), jnp.int32),
        ],
        scratch_shapes=[
            pltpu.VMEM((NBUF * CHUNK, EMBED_DIM), jnp.float32),
            pltpu.VMEM((BATCH, EMBED_DIM), jnp.float32),
            pltpu.SemaphoreType.DMA((NBUF,)),
            pltpu.SemaphoreType.DMA((NBUF,)),
            pltpu.SemaphoreType.DMA,
            pltpu.SemaphoreType.DMA((4,)),
        ],
    )(queue_ptr, embed_queue, embeddings)
    return new_queue, new_ptr


# speedup vs baseline: 1.0119x; 1.0019x over previous
import jax
import jax.numpy as jnp
from jax.experimental import pallas as pl
from jax.experimental.pallas import tpu as pltpu

SIZE = 262144
EMBED_DIM = 128
BATCH = 4096
CHUNK = 16384
NCHUNK = SIZE // CHUNK
NBUF = 7


def _body(ptr_ref, q_hbm, emb_hbm, out_hbm, optr_ref,
          bufs, emb_buf, sem_in, sem_out, sem_emb, sem_fill):
    raw_ptr = ptr_ref[0]
    ptr = jnp.clip(raw_ptr, 0, SIZE - BATCH)
    optr_ref[0] = (raw_ptr + BATCH) % SIZE

    k0 = ptr // CHUNK

    SCHEDULE = (
        [(2 * CHUNK, CHUNK), (3 * CHUNK, CHUNK)]
        + [(0, CHUNK), (CHUNK, CHUNK)]
        + [((4 + j) * CHUNK, CHUNK) for j in range(NCHUNK - 4)]
    )
    NPOS = len(SCHEDULE)
    WPOS = 2

    def npiece(p):
        return 4 if p in (0, NPOS - 1) else 1

    def hbm_start(p):
        rel, _ = SCHEDULE[p]
        return (k0 * CHUNK + rel) % SIZE

    def in_copy(p, k=0):
        _, n = SCHEDULE[p]
        split = npiece(p) > 1
        n //= npiece(p)
        s = p % NBUF
        sem = sem_fill.at[k] if split else sem_in.at[s]
        return pltpu.make_async_copy(
            q_hbm.at[pl.ds(hbm_start(p) + k * n, n)],
            bufs.at[pl.ds(s * CHUNK + k * n, n)], sem)

    def out_copy(p, k=0):
        _, n = SCHEDULE[p]
        n //= npiece(p)
        s = p % NBUF
        return pltpu.make_async_copy(
            bufs.at[pl.ds(s * CHUNK + k * n, n)],
            out_hbm.at[pl.ds(hbm_start(p) + k * n, n)], sem_out.at[s])

    emb_in = pltpu.make_async_copy(emb_hbm, emb_buf, sem_emb)
    off = ptr - k0 * CHUNK
    splice = pltpu.make_async_copy(
        emb_buf, bufs.at[pl.ds(WPOS * CHUNK + off, BATCH)], sem_emb)

    in_waited = set()
    out_waited = set()

    def in_half(p, h):
        n = CHUNK // 2
        s = p % NBUF
        return pltpu.make_async_copy(
            q_hbm.at[pl.ds(hbm_start(p) + h * n, n)],
            bufs.at[pl.ds(s * CHUNK + h * n, n)], sem_in.at[s])

    def start_in(p):
        if npiece(p) > 1:
            for k in range(npiece(p)):
                in_copy(p, k).start()
        else:
            in_half(p, 0).start()
            in_half(p, 1).start()

    def ensure_in(p):
        if p not in in_waited:
            if npiece(p) > 1:
                for k in range(npiece(p)):
                    in_copy(p, k).wait()
            else:
                in_half(p, 0).wait()
                in_half(p, 1).wait()
            in_waited.add(p)

    def ensure_out(p):
        if p not in out_waited:
            for k in range(npiece(p)):
                out_copy(p, k).wait()
            out_waited.add(p)

    emb_in.start()
    for p in range(NBUF):
        start_in(p)

    for p in range(NPOS):
        if npiece(p) == 1:
            ensure_in(p)
            if p == WPOS:
                ensure_in(p + 1)
                emb_in.wait()
                splice.start()
                splice.wait()
            out_copy(p).start()
        else:
            for k in range(npiece(p)):
                in_copy(p, k).wait()
                out_copy(p, k).start()
            in_waited.add(p)
        nxt = p + NBUF
        if nxt < NPOS:
            ensure_out(nxt - NBUF)
            start_in(nxt)
    for p in range(NPOS):
        ensure_out(p)


def kernel(embed_queue, queue_ptr, embeddings):
    new_queue, new_ptr = pl.pallas_call(
        _body,
        in_specs=[
            pl.BlockSpec(memory_space=pltpu.SMEM),
            pl.BlockSpec(memory_space=pl.ANY),
            pl.BlockSpec(memory_space=pl.ANY),
        ],
        out_specs=[
            pl.BlockSpec(memory_space=pl.ANY),
            pl.BlockSpec(memory_space=pltpu.SMEM),
        ],
        out_shape=[
            jax.ShapeDtypeStruct((SIZE, EMBED_DIM), jnp.float32),
            jax.ShapeDtypeStruct((1,), jnp.int32),
        ],
        scratch_shapes=[
            pltpu.VMEM((NBUF * CHUNK, EMBED_DIM), jnp.float32),
            pltpu.VMEM((BATCH, EMBED_DIM), jnp.float32),
            pltpu.SemaphoreType.DMA((NBUF,)),
            pltpu.SemaphoreType.DMA((NBUF,)),
            pltpu.SemaphoreType.DMA,
            pltpu.SemaphoreType.DMA((4,)),
        ],
    )(queue_ptr, embed_queue, embeddings)
    return new_queue, new_ptr
